# baseline (speedup 1.0000x reference)
import jax
import jax.numpy as jnp
from jax.experimental import pallas as pl

BLK_D = 512


def _add_body(x_ref, lut_ref, o_ref):
    o_ref[...] = x_ref[...] + lut_ref[...]


def kernel(x, lut_weight):
    B, P, D = x.shape
    grid = (D // BLK_D, B)
    return pl.pallas_call(
        _add_body,
        grid=grid,
        in_specs=[
            pl.BlockSpec((1, P, BLK_D), lambda i, j: (j, 0, i)),
            pl.BlockSpec((P, BLK_D), lambda i, j: (0, i)),
        ],
        out_specs=pl.BlockSpec((1, P, BLK_D), lambda i, j: (j, 0, i)),
        out_shape=jax.ShapeDtypeStruct((B, P, D), x.dtype),
    )(x, lut_weight)
